# Initial kernel scaffold; baseline (speedup 1.0000x reference)
import jax
import jax.numpy as jnp
from jax.experimental import pallas as pl


def kernel(vectors, node_specie, senders, receivers, embed, W_r, W_mix, W_sp, W_read):
    raise NotImplementedError("write your pallas kernel here")


# speedup vs baseline: 23.7956x; 23.7956x over previous
import jax
import jax.numpy as jnp
from jax import lax
from jax.experimental import pallas as pl
from jax.experimental.pallas import tpu as pltpu
from jax.experimental.pallas import tpu_sc as plsc

N = 10000
E = 320000
C = 128
S = 5
NB = 8
L = 2
RMAX = 5.0
AVG = 32.0
SH = 4

NP = 10240

NC, NS = 2, 16
NW = NC * NS
EW = E // NW
CH = 80
NCH = EW // CH
RT = NP // NS
ZR = 160

import functools


@functools.cache
def _sc_mesh():
    return plsc.VectorSubcoreMesh(
        core_axis_name="c", subcore_axis_name="s",
        num_cores=NC, num_subcores=NS)


def _gather_body(feats_hbm, senders_hbm, g_hbm, idx_v, rows_v, sem):
    cid = lax.axis_index("c")
    sid = lax.axis_index("s")
    wid = sid * NC + cid

    def body(k, carry):
        base = wid * EW + k * CH
        pltpu.sync_copy(senders_hbm.at[pl.ds(base, CH)], idx_v)
        pltpu.async_copy(feats_hbm.at[idx_v], rows_v, sem).wait()
        pltpu.sync_copy(rows_v, g_hbm.at[pl.ds(base, CH), :])
        return carry

    lax.fori_loop(0, NCH, body, 0)


@functools.cache
def _gather_kernel():
    return pl.kernel(
        _gather_body,
        out_type=jax.ShapeDtypeStruct((E, C), jnp.float32),
        mesh=_sc_mesh(),
        scratch_types=[
            pltpu.VMEM((CH,), jnp.int32),
            pltpu.VMEM((CH, C), jnp.float32),
            pltpu.SemaphoreType.DMA,
        ],
    )


def _scatter_body(q_hbm, recv_hbm, zeros_hbm, acc2_hbm,
                  idx_v, rows_v, stage_v, acc_sh, sem):
    cid = lax.axis_index("c")
    sid = lax.axis_index("s")
    wid = sid * NC + cid
    row0 = sid * RT

    pltpu.sync_copy(zeros_hbm, stage_v)
    for j in range(RT // ZR):
        pltpu.sync_copy(stage_v, acc_sh.at[pl.ds(row0 + j * ZR, ZR), :])
    plsc.subcore_barrier()

    def body(k, carry):
        base = wid * EW + k * CH
        pltpu.sync_copy(recv_hbm.at[pl.ds(base, CH)], idx_v)
        pltpu.sync_copy(q_hbm.at[pl.ds(base, CH), :], rows_v)
        pltpu.sync_copy(rows_v, acc_sh.at[idx_v], add=True)
        return carry

    lax.fori_loop(0, NCH, body, 0)
    plsc.subcore_barrier()

    for j in range(RT // ZR):
        r0 = row0 + j * ZR
        pltpu.sync_copy(acc_sh.at[pl.ds(r0, ZR), :], stage_v)
        pltpu.sync_copy(stage_v, acc2_hbm.at[cid, pl.ds(r0, ZR), :])


@functools.cache
def _scatter_kernel():
    return pl.kernel(
        _scatter_body,
        out_type=jax.ShapeDtypeStruct((NC, NP, C), jnp.float32),
        mesh=_sc_mesh(),
        scratch_types=[
            pltpu.VMEM((CH,), jnp.int32),
            pltpu.VMEM((CH, C), jnp.float32),
            pltpu.VMEM((ZR, C), jnp.float32),
            pltpu.VMEM_SHARED((NP, C), jnp.float32),
            pltpu.SemaphoreType.DMA,
        ],
    )


BE = 2560
BN = 1024


def _edge_body(v_ref, g_ref, wr_ref, wmix_ref, q_ref):
    v = v_ref[...]
    r2 = jnp.sum(v * v, axis=1, keepdims=True) + 1e-12
    r = jnp.sqrt(r2)
    inv_r = 1.0 / r
    u = v * inv_r
    nvec = (lax.broadcasted_iota(jnp.int32, (1, NB), 1) + 1).astype(jnp.float32)
    arg = (r * (jnp.pi / RMAX)) * nvec
    bessel = jnp.sqrt(2.0 / RMAX) * jnp.sin(arg) * inv_r
    x = r * (1.0 / RMAX)
    x5 = x * x * x * x * x
    env = jnp.where(x < 1.0, 1.0 - 21.0 * x5 + 35.0 * x5 * x - 15.0 * x5 * x * x, 0.0)
    radial = bessel * env
    rw_lin = jnp.dot(radial, wr_ref[...], preferred_element_type=jnp.float32)
    rw = rw_lin * jax.nn.sigmoid(rw_lin)
    m = g_ref[...] * rw
    q = jnp.dot(m, wmix_ref[0], preferred_element_type=jnp.float32)
    for s in range(SH - 1):
        q = q + jnp.dot(m, wmix_ref[s + 1],
                        preferred_element_type=jnp.float32) * u[:, s:s + 1]
    q_ref[...] = q


def _edge_tc(vpad, g, wr, wmix4):
    return pl.pallas_call(
        _edge_body,
        grid=(E // BE,),
        in_specs=[
            pl.BlockSpec((BE, 4), lambda i: (i, 0)),
            pl.BlockSpec((BE, C), lambda i: (i, 0)),
            pl.BlockSpec((NB, C), lambda i: (0, 0)),
            pl.BlockSpec((SH, C, C), lambda i: (0, 0, 0)),
        ],
        out_specs=pl.BlockSpec((BE, C), lambda i: (i, 0)),
        out_shape=jax.ShapeDtypeStruct((E, C), jnp.float32),
    )(vpad, g, wr, wmix4)


def _node_body(acc_ref, oh_ref, wsp_ref, wread_ref, feats_ref, out_ref):
    a = (acc_ref[0] + acc_ref[1]) * (1.0 / AVG)
    hs = jnp.dot(oh_ref[...], wsp_ref[...], preferred_element_type=jnp.float32)
    h = a * hs
    feats = h + 0.5 * h * h + (1.0 / 6.0) * h * h * h
    feats_ref[...] = feats
    out_ref[...] = jnp.sum(feats * wread_ref[...], axis=1, keepdims=True)


def _node_tc(acc2, oh, wsp, wread):
    return pl.pallas_call(
        _node_body,
        grid=(NP // BN,),
        in_specs=[
            pl.BlockSpec((NC, BN, C), lambda i: (0, i, 0)),
            pl.BlockSpec((BN, 8), lambda i: (i, 0)),
            pl.BlockSpec((8, C), lambda i: (0, 0)),
            pl.BlockSpec((1, C), lambda i: (0, 0)),
        ],
        out_specs=[
            pl.BlockSpec((BN, C), lambda i: (i, 0)),
            pl.BlockSpec((BN, 1), lambda i: (i, 0)),
        ],
        out_shape=[
            jax.ShapeDtypeStruct((NP, C), jnp.float32),
            jax.ShapeDtypeStruct((NP, 1), jnp.float32),
        ],
    )(acc2, oh, wsp, wread)


def _embed_body(oh_ref, emb_ref, f_ref):
    f_ref[...] = jnp.dot(oh_ref[...], emb_ref[...],
                         preferred_element_type=jnp.float32)


def _embed_tc(oh, emb8):
    return pl.pallas_call(
        _embed_body,
        grid=(NP // BN,),
        in_specs=[
            pl.BlockSpec((BN, 8), lambda i: (i, 0)),
            pl.BlockSpec((8, C), lambda i: (0, 0)),
        ],
        out_specs=pl.BlockSpec((BN, C), lambda i: (i, 0)),
        out_shape=jax.ShapeDtypeStruct((NP, C), jnp.float32),
    )(oh, emb8)


def kernel(vectors, node_specie, senders, receivers, embed, W_r, W_mix, W_sp, W_read):
    f32 = jnp.float32
    vpad = jnp.concatenate([vectors, jnp.zeros((E, 1), f32)], axis=1)
    senders = senders.astype(jnp.int32)
    receivers = receivers.astype(jnp.int32)
    sp = jnp.concatenate(
        [node_specie.astype(jnp.int32), jnp.zeros((NP - N,), jnp.int32)])
    oh = (sp[:, None] == jnp.arange(8, dtype=jnp.int32)[None, :]).astype(f32)
    emb8 = jnp.zeros((8, C), f32).at[:S].set(embed)
    wsp8 = jnp.zeros((L, 8, C), f32).at[:, :S].set(W_sp)
    wmix = W_mix.reshape(L, C, SH, C).transpose(0, 2, 1, 3)
    wread = W_read.transpose(0, 2, 1)
    zeros_zr = jnp.zeros((ZR, C), f32)

    feats = _embed_tc(oh, emb8)
    outs = []
    for l in range(L):
        g = _gather_kernel()(feats, senders)
        q = _edge_tc(vpad, g, W_r[l], wmix[l])
        acc2 = _scatter_kernel()(q, receivers, zeros_zr)
        feats, o = _node_tc(acc2, oh, wsp8[l], wread[l])
        outs.append(o)
    return jnp.concatenate(outs, axis=1)[:N]
